# baseline (speedup 1.0000x reference)
import jax
import jax.numpy as jnp
from jax import lax
from jax.experimental import pallas as pl
from jax.experimental.pallas import tpu as pltpu

N = 8192
D_IN = 768
E = 8
HID = 256
D_OUT = 256

BT2 = 4096
NT = N // BT2


def _gelu(h):
    return 0.5 * h * (1.0 + lax.erf(h * (2.0 ** -0.5)))


def _ln_mxu(h):
    avg = jnp.full((HID, HID), 1.0 / HID, dtype=jnp.bfloat16)
    mu = jnp.dot(h.astype(jnp.bfloat16), avg,
                 preferred_element_type=jnp.float32)
    d = h - mu
    var = jnp.dot((d * d).astype(jnp.bfloat16), avg,
                  preferred_element_type=jnp.float32)
    return d * lax.rsqrt(var + 1e-5)


def _body(x_ref, wg1_ref, wg2_ref, W1_ref, b1_ref, g1_ref, be1_ref,
          W2_ref, b2_ref, g2_ref, be2_ref, W3_ref, b3_ref, o_ref, w_scr):
    e = pl.program_id(1)

    @pl.when(e == 0)
    def _():
        t = jnp.tanh(jnp.dot(x_ref[...], wg1_ref[...],
                             preferred_element_type=jnp.float32))
        logits = jnp.dot(t, wg2_ref[...], preferred_element_type=jnp.float32)
        m = jnp.max(logits, axis=-1, keepdims=True)
        ex = jnp.exp(logits - m)
        gw = ex / jnp.sum(ex, axis=-1, keepdims=True)
        a1 = jnp.argmax(gw, axis=-1)
        m1 = jnp.max(gw, axis=-1)
        lane = lax.broadcasted_iota(jnp.int32, gw.shape, 1)
        gw2 = jnp.where(lane == a1[:, None], -1.0, gw)
        a2 = jnp.argmax(gw2, axis=-1)
        m2 = jnp.max(gw2, axis=-1)
        s = m1 + m2 + 1e-12
        w_scr[...] = (jnp.where(lane == a1[:, None], (m1 / s)[:, None], 0.0)
                      + jnp.where(lane == a2[:, None], (m2 / s)[:, None], 0.0))

    @pl.when(e > 0)
    def _():
        xb = x_ref[...].astype(jnp.bfloat16)
        h = jnp.dot(xb, W1_ref[0].astype(jnp.bfloat16),
                    preferred_element_type=jnp.float32) + b1_ref[0]
        h = _ln_mxu(h) * g1_ref[0] + be1_ref[0]
        h = _gelu(h)
        h = jnp.dot(h.astype(jnp.bfloat16), W2_ref[0].astype(jnp.bfloat16),
                    preferred_element_type=jnp.float32) + b2_ref[0]
        h = _ln_mxu(h) * g2_ref[0] + be2_ref[0]
        h = _gelu(h)
        h = jnp.dot(h.astype(jnp.bfloat16), W3_ref[0].astype(jnp.bfloat16),
                    preferred_element_type=jnp.float32) + b3_ref[0]
        wl = lax.broadcasted_iota(jnp.int32, (BT2, E), 1)
        wcol = jnp.sum(jnp.where(wl == e - 1, w_scr[...], 0.0),
                       axis=1, keepdims=True)
        contrib = h * wcol

        @pl.when(e == 1)
        def _():
            o_ref[...] = contrib

        @pl.when(e > 1)
        def _():
            o_ref[...] = o_ref[...] + contrib


def kernel(x, Wg1, Wg2, W1, b1, g1, be1, W2, b2, g2, be2, W3, b3):
    r3 = lambda a: a.reshape(E, 1, a.shape[-1])
    ei = lambda e: jnp.maximum(e - 1, 0)
    we = lambda s: pl.BlockSpec(
        (1,) + s, lambda i, e, _n=len(s): (ei(e),) + (0,) * _n)
    return pl.pallas_call(
        _body,
        grid=(NT, E + 1),
        in_specs=[
            pl.BlockSpec((BT2, D_IN), lambda i, e: (i, 0)),
            pl.BlockSpec((D_IN, 2 * E), lambda i, e: (0, 0)),
            pl.BlockSpec((2 * E, E), lambda i, e: (0, 0)),
            we((D_IN, HID)), we((1, HID)), we((1, HID)), we((1, HID)),
            we((HID, HID)), we((1, HID)), we((1, HID)), we((1, HID)),
            we((HID, D_OUT)), we((1, D_OUT)),
        ],
        out_specs=pl.BlockSpec((BT2, D_OUT), lambda i, e: (i, 0)),
        out_shape=jax.ShapeDtypeStruct((N, D_OUT), jnp.float32),
        scratch_shapes=[pltpu.VMEM((BT2, E), jnp.float32)],
    )(x, Wg1, Wg2, W1, r3(b1), r3(g1), r3(be1), W2, r3(b2), r3(g2), r3(be2),
      W3, r3(b3))


# speedup vs baseline: 1.1429x; 1.1429x over previous
import jax
import jax.numpy as jnp
from jax import lax
from jax.experimental import pallas as pl
from jax.experimental.pallas import tpu as pltpu

N = 8192
D_IN = 768
E = 8
HID = 256
D_OUT = 256

BT2 = 4096
NT = N // BT2


def _gelu(h):
    return 0.5 * h * (1.0 + lax.erf(h * (2.0 ** -0.5)))


def _ln(h):
    mu = jnp.mean(h, axis=-1, keepdims=True)
    var = jnp.mean((h - mu) ** 2, axis=-1, keepdims=True)
    return (h - mu) * lax.rsqrt(var + 1e-5)


def _body(x_ref, wg1_ref, wg2_ref, W1_ref, b1_ref, g1_ref, be1_ref,
          W2_ref, b2_ref, g2_ref, be2_ref, W3_ref, b3_ref, o_ref, w_scr):
    e = pl.program_id(1)

    @pl.when(e == 0)
    def _():
        t = jnp.tanh(jnp.dot(x_ref[...], wg1_ref[...],
                             preferred_element_type=jnp.float32))
        logits = jnp.dot(t, wg2_ref[...], preferred_element_type=jnp.float32)
        m = jnp.max(logits, axis=-1, keepdims=True)
        ex = jnp.exp(logits - m)
        gw = ex / jnp.sum(ex, axis=-1, keepdims=True)
        a1 = jnp.argmax(gw, axis=-1)
        m1 = jnp.max(gw, axis=-1)
        lane = lax.broadcasted_iota(jnp.int32, gw.shape, 1)
        gw2 = jnp.where(lane == a1[:, None], -1.0, gw)
        a2 = jnp.argmax(gw2, axis=-1)
        m2 = jnp.max(gw2, axis=-1)
        s = m1 + m2 + 1e-12
        w_scr[...] = (jnp.where(lane == a1[:, None], (m1 / s)[:, None], 0.0)
                      + jnp.where(lane == a2[:, None], (m2 / s)[:, None], 0.0))

    @pl.when(e > 0)
    def _():
        xb = x_ref[...].astype(jnp.bfloat16)
        h = jnp.dot(xb, W1_ref[0].astype(jnp.bfloat16),
                    preferred_element_type=jnp.float32) + b1_ref[0]
        h = _ln(h) * g1_ref[0] + be1_ref[0]
        h = _gelu(h)
        h = jnp.dot(h.astype(jnp.bfloat16), W2_ref[0].astype(jnp.bfloat16),
                    preferred_element_type=jnp.float32) + b2_ref[0]
        h = _ln(h) * g2_ref[0] + be2_ref[0]
        h = _gelu(h)
        h = jnp.dot(h.astype(jnp.bfloat16), W3_ref[0].astype(jnp.bfloat16),
                    preferred_element_type=jnp.float32) + b3_ref[0]
        wl = lax.broadcasted_iota(jnp.int32, (BT2, E), 1)
        wcol = jnp.sum(jnp.where(wl == e - 1, w_scr[...], 0.0),
                       axis=1, keepdims=True)
        contrib = h * wcol

        @pl.when(e == 1)
        def _():
            o_ref[...] = contrib

        @pl.when(e > 1)
        def _():
            o_ref[...] = o_ref[...] + contrib


def kernel(x, Wg1, Wg2, W1, b1, g1, be1, W2, b2, g2, be2, W3, b3):
    r3 = lambda a: a.reshape(E, 1, a.shape[-1])
    ei = lambda e: jnp.maximum(e - 1, 0)
    we = lambda s: pl.BlockSpec(
        (1,) + s, lambda i, e, _n=len(s): (ei(e),) + (0,) * _n)
    return pl.pallas_call(
        _body,
        grid=(NT, E + 1),
        in_specs=[
            pl.BlockSpec((BT2, D_IN), lambda i, e: (i, 0)),
            pl.BlockSpec((D_IN, 2 * E), lambda i, e: (0, 0)),
            pl.BlockSpec((2 * E, E), lambda i, e: (0, 0)),
            we((D_IN, HID)), we((1, HID)), we((1, HID)), we((1, HID)),
            we((HID, HID)), we((1, HID)), we((1, HID)), we((1, HID)),
            we((HID, D_OUT)), we((1, D_OUT)),
        ],
        out_specs=pl.BlockSpec((BT2, D_OUT), lambda i, e: (i, 0)),
        out_shape=jax.ShapeDtypeStruct((N, D_OUT), jnp.float32),
        scratch_shapes=[pltpu.VMEM((BT2, E), jnp.float32)],
    )(x, Wg1, Wg2, W1, r3(b1), r3(g1), r3(be1), W2, r3(b2), r3(g2), r3(be2),
      W3, r3(b3))
